# baseline (speedup 1.0000x reference)
import functools

import jax
import jax.numpy as jnp
from jax import lax
from jax.experimental import pallas as pl
from jax.experimental.pallas import tpu as pltpu
from jax.experimental.pallas import tpu_sc as plsc

N = 10000
E = 320000
D = 128
NC = 2
NS = 16
NW = NC * NS
K = 128
NCH = 80
EPWP = NCH * K
EPAD = NW * EPWP
NP = 10240
RPW = NP // NS


@functools.cache
def _sc_mesh():
  return plsc.VectorSubcoreMesh(core_axis_name="c", subcore_axis_name="s")


G = 8
NG = NCH // G


def _sc_agg(h, src4, dst4, zeros_rows):
  @functools.partial(
      pl.kernel,
      mesh=_sc_mesh(),
      out_type=jax.ShapeDtypeStruct((NC * NP, D), jnp.float32),
      scratch_types=[
          pltpu.VMEM((G, K), jnp.int32),
          pltpu.VMEM((G, K), jnp.int32),
          pltpu.VMEM((G, K), jnp.int32),
          pltpu.VMEM((G, K), jnp.int32),
          pltpu.VMEM((K, D), jnp.float32),
          pltpu.VMEM((K, D), jnp.float32),
          pltpu.VMEM_SHARED((NP, D), jnp.float32),
          pltpu.SemaphoreType.DMA,
          pltpu.SemaphoreType.DMA,
          pltpu.SemaphoreType.DMA,
          pltpu.SemaphoreType.DMA,
          pltpu.SemaphoreType.DMA,
          pltpu.SemaphoreType.DMA,
          pltpu.SemaphoreType.DMA,
      ],
  )
  def body(h_hbm, src_hbm, dst_hbm, zr_hbm, agg_out, srcb0, srcb1,
           dstb0, dstb1, rows0, rows1, acc_sh, sg0, sg1, ss0, ss1,
           si0, si1, sst):
    cid = lax.axis_index("c")
    sid = lax.axis_index("s")
    wid = cid * NS + sid
    row0 = sid * RPW
    srcb = (srcb0, srcb1)
    dstb = (dstb0, dstb1)
    rows = (rows0, rows1)
    sg = (sg0, sg1)
    ss = (ss0, ss1)
    si = (si0, si1)

    pltpu.async_copy(zr_hbm, rows0, sst).wait()

    @pl.loop(0, RPW, step=K)
    def _(j):
      pltpu.sync_copy(rows0, acc_sh.at[pl.ds(row0 + j, K)])

    plsc.subcore_barrier()

    pltpu.async_copy(src_hbm.at[wid, 0], srcb0, sst).wait()
    pltpu.async_copy(dst_hbm.at[wid, 0], dstb0, sst).wait()
    pltpu.async_copy(h_hbm.at[srcb0.at[0]], rows0, sg0)

    def chunk(g, gb, j):
      b = j % 2
      ob = 1 - b
      pltpu.make_async_copy(h_hbm.at[srcb0.at[0]], rows[b], sg[b]).wait()
      pltpu.async_copy(rows[b], acc_sh.at[dstb[gb].at[j]], ss[b], add=True)
      if j == 0:
        @pl.when(g >= 1)
        def _():
          pltpu.make_async_copy(h_hbm.at[srcb0.at[0]], rows[ob],
                                ss[ob]).wait()
        @pl.when(g + 1 < NG)
        def _():
          pltpu.async_copy(src_hbm.at[wid, g + 1], srcb[1 - gb], si[1 - gb])
          pltpu.async_copy(dst_hbm.at[wid, g + 1], dstb[1 - gb], si[1 - gb])
      else:
        pltpu.make_async_copy(h_hbm.at[srcb0.at[0]], rows[ob], ss[ob]).wait()
      if j < G - 1:
        pltpu.async_copy(h_hbm.at[srcb[gb].at[j + 1]], rows[ob], sg[ob])
      else:
        @pl.when(g + 1 < NG)
        def _():
          pltpu.make_async_copy(src_hbm.at[wid, 0], srcb[1 - gb],
                                si[1 - gb]).wait()
          pltpu.make_async_copy(dst_hbm.at[wid, 0], dstb[1 - gb],
                                si[1 - gb]).wait()
          pltpu.async_copy(h_hbm.at[srcb[1 - gb].at[0]], rows[ob], sg[ob])

    @pl.loop(0, NG, step=2)
    def _(go):
      for gb in (0, 1):
        for j in range(G):
          chunk(go + gb, gb, j)

    pltpu.make_async_copy(h_hbm.at[srcb0.at[0]], rows[(G - 1) % 2],
                          ss[(G - 1) % 2]).wait()

    plsc.subcore_barrier()
    out_r = cid * NP + row0

    @pl.loop(0, RPW, step=K)
    def _(j):
      pltpu.sync_copy(acc_sh.at[pl.ds(row0 + j, K)], rows0)
      pltpu.sync_copy(rows0, agg_out.at[pl.ds(out_r + j, K)])

  return body(h, src4, dst4, zeros_rows).reshape(NC, NP, D)


def _sc_deg(dst3, zeros_rows, ones_rows):
  NQ = 8

  @functools.partial(
      pl.kernel,
      mesh=_sc_mesh(),
      out_type=jax.ShapeDtypeStruct((NC * NP, D), jnp.float32),
      scratch_types=[
          pltpu.VMEM((NCH, K), jnp.int32),
          pltpu.VMEM((K, D), jnp.float32),
          pltpu.VMEM_SHARED((NP, D), jnp.float32),
          pltpu.SemaphoreType.DMA,
          pltpu.SemaphoreType.DMA,
      ],
  )
  def body(dst_hbm, zr_hbm, ones_hbm, deg_out, dst_v, rows_v, acc_sh,
           ssc, sst):
    cid = lax.axis_index("c")
    sid = lax.axis_index("s")
    wid = cid * NS + sid
    row0 = sid * RPW
    pltpu.async_copy(dst_hbm.at[wid], dst_v, sst).wait()
    pltpu.async_copy(zr_hbm, rows_v, sst).wait()

    @pl.loop(0, RPW, step=K)
    def _(j):
      pltpu.sync_copy(rows_v, acc_sh.at[pl.ds(row0 + j, K)])

    plsc.subcore_barrier()
    pltpu.async_copy(ones_hbm, rows_v, sst).wait()

    @pl.loop(0, NQ)
    def _(cc):
      pltpu.async_copy(rows_v, acc_sh.at[dst_v.at[cc]], ssc, add=True)

    @pl.loop(NQ, NCH)
    def _(cc):
      pltpu.make_async_copy(ones_hbm, rows_v, ssc).wait()
      pltpu.async_copy(rows_v, acc_sh.at[dst_v.at[cc]], ssc, add=True)

    @pl.loop(0, NQ)
    def _(cc):
      pltpu.make_async_copy(ones_hbm, rows_v, ssc).wait()

    plsc.subcore_barrier()
    out_r = cid * NP + row0

    @pl.loop(0, RPW, step=K)
    def _(j):
      pltpu.sync_copy(acc_sh.at[pl.ds(row0 + j, K)], rows_v)
      pltpu.sync_copy(rows_v, deg_out.at[pl.ds(out_r + j, K)])

  return body(dst3, zeros_rows, ones_rows).reshape(NC, NP, D)


ROWS_BLK = 1000


def _tc_layer1_body(h_ref, agg_ref, deg_ref, ws_ref, wn_ref, b_ref, out_ref):
  agg = agg_ref[0] + agg_ref[1]
  deg = deg_ref[0] + deg_ref[1]
  dinv = 1.0 / jnp.maximum(deg, 1.0)
  hs = jnp.dot(h_ref[...], ws_ref[...], preferred_element_type=jnp.float32)
  hn = jnp.dot(agg, wn_ref[...], preferred_element_type=jnp.float32)
  out_ref[...] = jnp.maximum(hs + hn * dinv + b_ref[...], 0.0)


def _tc_layer1(h, aggp, degs, W_self, W_neigh, b):
  grid = (N // ROWS_BLK,)
  return pl.pallas_call(
      _tc_layer1_body,
      grid=grid,
      in_specs=[
          pl.BlockSpec((ROWS_BLK, D), lambda i: (i, 0)),
          pl.BlockSpec((NC, ROWS_BLK, D), lambda i: (0, i, 0)),
          pl.BlockSpec((NC, ROWS_BLK, 1), lambda i: (0, i, 0)),
          pl.BlockSpec((D, D), lambda i: (0, 0)),
          pl.BlockSpec((D, D), lambda i: (0, 0)),
          pl.BlockSpec((1, D), lambda i: (0, 0)),
      ],
      out_specs=pl.BlockSpec((ROWS_BLK, D), lambda i: (i, 0)),
      out_shape=jax.ShapeDtypeStruct((N, D), jnp.float32),
  )(h, aggp, degs, W_self, W_neigh, b.reshape(1, D))


def _tc_layer2_body(h_ref, agg_ref, deg_ref, ws_ref, wn_ref, b_ref,
                    wr0_ref, br0_ref, wr1_ref, br1_ref, out_ref):
  agg = agg_ref[0] + agg_ref[1]
  deg = deg_ref[0] + deg_ref[1]
  dinv = 1.0 / jnp.maximum(deg, 1.0)
  hs = jnp.dot(h_ref[...], ws_ref[...], preferred_element_type=jnp.float32)
  hn = jnp.dot(agg, wn_ref[...], preferred_element_type=jnp.float32)
  h2 = jnp.maximum(hs + hn * dinv + b_ref[...], 0.0)
  r = jnp.maximum(
      jnp.dot(h2, wr0_ref[...], preferred_element_type=jnp.float32)
      + br0_ref[...], 0.0)
  out_ref[...] = (
      jnp.dot(r, wr1_ref[...], preferred_element_type=jnp.float32)
      + br1_ref[...])


def _tc_layer2(h, aggp, degs, W_self, W_neigh, b, Wr0, br0, Wr1, br1):
  grid = (N // ROWS_BLK,)
  H1 = Wr0.shape[1]
  return pl.pallas_call(
      _tc_layer2_body,
      grid=grid,
      in_specs=[
          pl.BlockSpec((ROWS_BLK, D), lambda i: (i, 0)),
          pl.BlockSpec((NC, ROWS_BLK, D), lambda i: (0, i, 0)),
          pl.BlockSpec((NC, ROWS_BLK, 1), lambda i: (0, i, 0)),
          pl.BlockSpec((D, D), lambda i: (0, 0)),
          pl.BlockSpec((D, D), lambda i: (0, 0)),
          pl.BlockSpec((1, D), lambda i: (0, 0)),
          pl.BlockSpec((D, H1), lambda i: (0, 0)),
          pl.BlockSpec((1, H1), lambda i: (0, 0)),
          pl.BlockSpec((H1, 1), lambda i: (0, 0)),
          pl.BlockSpec((1, 1), lambda i: (0, 0)),
      ],
      out_specs=pl.BlockSpec((ROWS_BLK, 1), lambda i: (i, 0)),
      out_shape=jax.ShapeDtypeStruct((N, 1), jnp.float32),
  )(h, aggp, degs, W_self, W_neigh, b.reshape(1, D),
    Wr0, br0.reshape(1, H1), Wr1, br1.reshape(1, 1))


def kernel(x, edge_index, W_self0, W_neigh0, b0, W_self1, W_neigh1, b1,
           Wr0, br0, Wr1, br1):
  ei = edge_index.astype(jnp.int32)
  src4 = jnp.concatenate(
      [ei[0], jnp.zeros((EPAD - E,), jnp.int32)]).reshape(NW, NG, G, K)
  dst4 = jnp.concatenate(
      [ei[1], jnp.full((EPAD - E,), N, jnp.int32)]).reshape(NW, NG, G, K)
  dst3 = dst4.reshape(NW, NCH, K)
  zeros_rows = jnp.zeros((K, D), jnp.float32)
  ones_rows = jnp.ones((K, D), jnp.float32)

  degp = _sc_deg(dst3, zeros_rows, ones_rows)
  degs = degp[:, :, :1]
  x_dep = x + 0.0 * degs[0, :N]
  agg0 = _sc_agg(x_dep, src4, dst4, zeros_rows)
  h1 = _tc_layer1(x, agg0, degs, W_self0, W_neigh0, b0)
  agg1 = _sc_agg(h1, src4, dst4, zeros_rows)
  return _tc_layer2(h1, agg1, degs, W_self1, W_neigh1, b1, Wr0, br0, Wr1, br1)


# speedup vs baseline: 1.0005x; 1.0005x over previous
import functools

import jax
import jax.numpy as jnp
from jax import lax
from jax.experimental import pallas as pl
from jax.experimental.pallas import tpu as pltpu
from jax.experimental.pallas import tpu_sc as plsc

N = 10000
E = 320000
D = 128
NC = 2
NS = 16
NW = NC * NS
K = 128
NCH = 80
EPWP = NCH * K
EPAD = NW * EPWP
NP = 10240
RPW = NP // NS


@functools.cache
def _sc_mesh():
  return plsc.VectorSubcoreMesh(core_axis_name="c", subcore_axis_name="s")


G = 8
NG = NCH // G


def _sc_agg(h, src4, dst4, zeros_rows):
  @functools.partial(
      pl.kernel,
      mesh=_sc_mesh(),
      out_type=jax.ShapeDtypeStruct((NC * NP, D), jnp.float32),
      scratch_types=[
          pltpu.VMEM((G, K), jnp.int32),
          pltpu.VMEM((G, K), jnp.int32),
          pltpu.VMEM((G, K), jnp.int32),
          pltpu.VMEM((G, K), jnp.int32),
          pltpu.VMEM((K, D), jnp.float32),
          pltpu.VMEM((K, D), jnp.float32),
          pltpu.VMEM_SHARED((NP, D), jnp.float32),
          pltpu.SemaphoreType.DMA,
          pltpu.SemaphoreType.DMA,
          pltpu.SemaphoreType.DMA,
          pltpu.SemaphoreType.DMA,
          pltpu.SemaphoreType.DMA,
          pltpu.SemaphoreType.DMA,
          pltpu.SemaphoreType.DMA,
      ],
  )
  def body(h_hbm, src_hbm, dst_hbm, zr_hbm, agg_out, srcb0, srcb1,
           dstb0, dstb1, rows0, rows1, acc_sh, sg0, sg1, ss0, ss1,
           si0, si1, sst):
    cid = lax.axis_index("c")
    sid = lax.axis_index("s")
    wid = cid * NS + sid
    row0 = sid * RPW
    srcb = (srcb0, srcb1)
    dstb = (dstb0, dstb1)
    rows = (rows0, rows1)
    sg = (sg0, sg1)
    ss = (ss0, ss1)
    si = (si0, si1)

    pltpu.async_copy(zr_hbm, rows0, sst).wait()

    @pl.loop(0, RPW, step=K)
    def _(j):
      pltpu.sync_copy(rows0, acc_sh.at[pl.ds(row0 + j, K)])

    plsc.subcore_barrier()

    pltpu.async_copy(src_hbm.at[wid, 0], srcb0, sst).wait()
    pltpu.async_copy(dst_hbm.at[wid, 0], dstb0, sst).wait()

    def chunk(g, gb, j):
      b = j % 2
      cc_wait_static = j >= 2

      def wait_prev_scatter():
        pltpu.make_async_copy(h_hbm.at[srcb0.at[0]], rows[b], ss[b]).wait()

      if cc_wait_static:
        wait_prev_scatter()
      else:
        @pl.when(g >= 1)
        def _():
          wait_prev_scatter()
      if j == 0:
        @pl.when(g >= 1)
        def _():
          pltpu.make_async_copy(src_hbm.at[wid, 0], srcb[gb], si[gb]).wait()
          pltpu.make_async_copy(dst_hbm.at[wid, 0], dstb[gb], si[gb]).wait()
      if j == 2:
        @pl.when(g + 1 < NG)
        def _():
          pltpu.async_copy(src_hbm.at[wid, g + 1], srcb[1 - gb], si[1 - gb])
          pltpu.async_copy(dst_hbm.at[wid, g + 1], dstb[1 - gb], si[1 - gb])
      pltpu.async_copy(h_hbm.at[srcb[gb].at[j]], rows[b], sg0).wait()
      pltpu.async_copy(rows[b], acc_sh.at[dstb[gb].at[j]], ss[b], add=True)

    @pl.loop(0, NG, step=2)
    def _(go):
      for gb in (0, 1):
        for j in range(G):
          chunk(go + gb, gb, j)

    pltpu.make_async_copy(h_hbm.at[srcb0.at[0]], rows0, ss0).wait()
    pltpu.make_async_copy(h_hbm.at[srcb0.at[0]], rows1, ss1).wait()

    plsc.subcore_barrier()
    out_r = cid * NP + row0

    @pl.loop(0, RPW, step=K)
    def _(j):
      pltpu.sync_copy(acc_sh.at[pl.ds(row0 + j, K)], rows0)
      pltpu.sync_copy(rows0, agg_out.at[pl.ds(out_r + j, K)])

  return body(h, src4, dst4, zeros_rows).reshape(NC, NP, D)


def _sc_deg(dst3, zeros_rows, ones_rows):
  NQ = 8

  @functools.partial(
      pl.kernel,
      mesh=_sc_mesh(),
      out_type=jax.ShapeDtypeStruct((NC * NP, D), jnp.float32),
      scratch_types=[
          pltpu.VMEM((NCH, K), jnp.int32),
          pltpu.VMEM((K, D), jnp.float32),
          pltpu.VMEM_SHARED((NP, D), jnp.float32),
          pltpu.SemaphoreType.DMA,
          pltpu.SemaphoreType.DMA,
      ],
  )
  def body(dst_hbm, zr_hbm, ones_hbm, deg_out, dst_v, rows_v, acc_sh,
           ssc, sst):
    cid = lax.axis_index("c")
    sid = lax.axis_index("s")
    wid = cid * NS + sid
    row0 = sid * RPW
    pltpu.async_copy(dst_hbm.at[wid], dst_v, sst).wait()
    pltpu.async_copy(zr_hbm, rows_v, sst).wait()

    @pl.loop(0, RPW, step=K)
    def _(j):
      pltpu.sync_copy(rows_v, acc_sh.at[pl.ds(row0 + j, K)])

    plsc.subcore_barrier()
    pltpu.async_copy(ones_hbm, rows_v, sst).wait()

    @pl.loop(0, NQ)
    def _(cc):
      pltpu.async_copy(rows_v, acc_sh.at[dst_v.at[cc]], ssc, add=True)

    @pl.loop(NQ, NCH)
    def _(cc):
      pltpu.make_async_copy(ones_hbm, rows_v, ssc).wait()
      pltpu.async_copy(rows_v, acc_sh.at[dst_v.at[cc]], ssc, add=True)

    @pl.loop(0, NQ)
    def _(cc):
      pltpu.make_async_copy(ones_hbm, rows_v, ssc).wait()

    plsc.subcore_barrier()
    out_r = cid * NP + row0

    @pl.loop(0, RPW, step=K)
    def _(j):
      pltpu.sync_copy(acc_sh.at[pl.ds(row0 + j, K)], rows_v)
      pltpu.sync_copy(rows_v, deg_out.at[pl.ds(out_r + j, K)])

  return body(dst3, zeros_rows, ones_rows).reshape(NC, NP, D)


ROWS_BLK = 1000


def _tc_layer1_body(h_ref, agg_ref, deg_ref, ws_ref, wn_ref, b_ref, out_ref):
  agg = agg_ref[0] + agg_ref[1]
  deg = deg_ref[0] + deg_ref[1]
  dinv = 1.0 / jnp.maximum(deg, 1.0)
  hs = jnp.dot(h_ref[...], ws_ref[...], preferred_element_type=jnp.float32)
  hn = jnp.dot(agg, wn_ref[...], preferred_element_type=jnp.float32)
  out_ref[...] = jnp.maximum(hs + hn * dinv + b_ref[...], 0.0)


def _tc_layer1(h, aggp, degs, W_self, W_neigh, b):
  grid = (N // ROWS_BLK,)
  return pl.pallas_call(
      _tc_layer1_body,
      grid=grid,
      in_specs=[
          pl.BlockSpec((ROWS_BLK, D), lambda i: (i, 0)),
          pl.BlockSpec((NC, ROWS_BLK, D), lambda i: (0, i, 0)),
          pl.BlockSpec((NC, ROWS_BLK, 1), lambda i: (0, i, 0)),
          pl.BlockSpec((D, D), lambda i: (0, 0)),
          pl.BlockSpec((D, D), lambda i: (0, 0)),
          pl.BlockSpec((1, D), lambda i: (0, 0)),
      ],
      out_specs=pl.BlockSpec((ROWS_BLK, D), lambda i: (i, 0)),
      out_shape=jax.ShapeDtypeStruct((N, D), jnp.float32),
  )(h, aggp, degs, W_self, W_neigh, b.reshape(1, D))


def _tc_layer2_body(h_ref, agg_ref, deg_ref, ws_ref, wn_ref, b_ref,
                    wr0_ref, br0_ref, wr1_ref, br1_ref, out_ref):
  agg = agg_ref[0] + agg_ref[1]
  deg = deg_ref[0] + deg_ref[1]
  dinv = 1.0 / jnp.maximum(deg, 1.0)
  hs = jnp.dot(h_ref[...], ws_ref[...], preferred_element_type=jnp.float32)
  hn = jnp.dot(agg, wn_ref[...], preferred_element_type=jnp.float32)
  h2 = jnp.maximum(hs + hn * dinv + b_ref[...], 0.0)
  r = jnp.maximum(
      jnp.dot(h2, wr0_ref[...], preferred_element_type=jnp.float32)
      + br0_ref[...], 0.0)
  out_ref[...] = (
      jnp.dot(r, wr1_ref[...], preferred_element_type=jnp.float32)
      + br1_ref[...])


def _tc_layer2(h, aggp, degs, W_self, W_neigh, b, Wr0, br0, Wr1, br1):
  grid = (N // ROWS_BLK,)
  H1 = Wr0.shape[1]
  return pl.pallas_call(
      _tc_layer2_body,
      grid=grid,
      in_specs=[
          pl.BlockSpec((ROWS_BLK, D), lambda i: (i, 0)),
          pl.BlockSpec((NC, ROWS_BLK, D), lambda i: (0, i, 0)),
          pl.BlockSpec((NC, ROWS_BLK, 1), lambda i: (0, i, 0)),
          pl.BlockSpec((D, D), lambda i: (0, 0)),
          pl.BlockSpec((D, D), lambda i: (0, 0)),
          pl.BlockSpec((1, D), lambda i: (0, 0)),
          pl.BlockSpec((D, H1), lambda i: (0, 0)),
          pl.BlockSpec((1, H1), lambda i: (0, 0)),
          pl.BlockSpec((H1, 1), lambda i: (0, 0)),
          pl.BlockSpec((1, 1), lambda i: (0, 0)),
      ],
      out_specs=pl.BlockSpec((ROWS_BLK, 1), lambda i: (i, 0)),
      out_shape=jax.ShapeDtypeStruct((N, 1), jnp.float32),
  )(h, aggp, degs, W_self, W_neigh, b.reshape(1, D),
    Wr0, br0.reshape(1, H1), Wr1, br1.reshape(1, 1))


def kernel(x, edge_index, W_self0, W_neigh0, b0, W_self1, W_neigh1, b1,
           Wr0, br0, Wr1, br1):
  ei = edge_index.astype(jnp.int32)
  src4 = jnp.concatenate(
      [ei[0], jnp.zeros((EPAD - E,), jnp.int32)]).reshape(NW, NG, G, K)
  dst4 = jnp.concatenate(
      [ei[1], jnp.full((EPAD - E,), N, jnp.int32)]).reshape(NW, NG, G, K)
  dst3 = dst4.reshape(NW, NCH, K)
  zeros_rows = jnp.zeros((K, D), jnp.float32)
  ones_rows = jnp.ones((K, D), jnp.float32)

  degp = _sc_deg(dst3, zeros_rows, ones_rows)
  degs = degp[:, :, :1]
  x_dep = x + 0.0 * degs[0, :N]
  agg0 = _sc_agg(x_dep, src4, dst4, zeros_rows)
  h1 = _tc_layer1(x, agg0, degs, W_self0, W_neigh0, b0)
  agg1 = _sc_agg(h1, src4, dst4, zeros_rows)
  return _tc_layer2(h1, agg1, degs, W_self1, W_neigh1, b1, Wr0, br0, Wr1, br1)


# speedup vs baseline: 1.0015x; 1.0009x over previous
import functools

import jax
import jax.numpy as jnp
from jax import lax
from jax.experimental import pallas as pl
from jax.experimental.pallas import tpu as pltpu
from jax.experimental.pallas import tpu_sc as plsc

N = 10000
E = 320000
D = 128
NC = 2
NS = 16
NW = NC * NS
K = 128
NCH = 80
EPWP = NCH * K
EPAD = NW * EPWP
NP = 10240
RPW = NP // NS


@functools.cache
def _sc_mesh():
  return plsc.VectorSubcoreMesh(core_axis_name="c", subcore_axis_name="s")


G = 8
NG = NCH // G


def _sc_agg(h, src4, dst4, zeros_rows):
  @functools.partial(
      pl.kernel,
      mesh=_sc_mesh(),
      out_type=jax.ShapeDtypeStruct((NC * NP, D), jnp.float32),
      scratch_types=[
          pltpu.VMEM((G, K), jnp.int32),
          pltpu.VMEM((G, K), jnp.int32),
          pltpu.VMEM((G, K), jnp.int32),
          pltpu.VMEM((G, K), jnp.int32),
          pltpu.VMEM((K, D), jnp.float32),
          pltpu.VMEM((K, D), jnp.float32),
          pltpu.VMEM_SHARED((NP, D), jnp.float32),
          pltpu.SemaphoreType.DMA,
          pltpu.SemaphoreType.DMA,
          pltpu.SemaphoreType.DMA,
          pltpu.SemaphoreType.DMA,
          pltpu.SemaphoreType.DMA,
          pltpu.SemaphoreType.DMA,
          pltpu.SemaphoreType.DMA,
      ],
  )
  def body(h_hbm, src_hbm, dst_hbm, zr_hbm, agg_out, srcb0, srcb1,
           dstb0, dstb1, rows0, rows1, acc_sh, sg0, sg1, ss0, ss1,
           si0, si1, sst):
    cid = lax.axis_index("c")
    sid = lax.axis_index("s")
    wid = cid * NS + sid
    row0 = sid * RPW
    srcb = (srcb0, srcb1)
    dstb = (dstb0, dstb1)
    rows = (rows0, rows1)
    sg = (sg0, sg1)
    ss = (ss0, ss1)
    si = (si0, si1)

    pltpu.async_copy(zr_hbm, rows0, sst).wait()

    @pl.loop(0, RPW, step=K)
    def _(j):
      pltpu.sync_copy(rows0, acc_sh.at[pl.ds(row0 + j, K)])

    plsc.subcore_barrier()

    pltpu.async_copy(src_hbm.at[wid, 0], srcb0, sst).wait()
    pltpu.async_copy(dst_hbm.at[wid, 0], dstb0, sst).wait()

    def chunk(g, gb, j):
      b = j % 2
      cc_wait_static = j >= 2

      def wait_prev_scatter():
        pltpu.make_async_copy(h_hbm.at[srcb0.at[0]], rows[b], ss[b]).wait()

      if cc_wait_static:
        wait_prev_scatter()
      else:
        @pl.when(g >= 1)
        def _():
          wait_prev_scatter()
      if j == 0:
        @pl.when(g >= 1)
        def _():
          pltpu.make_async_copy(src_hbm.at[wid, 0], srcb[gb], si[gb]).wait()
          pltpu.make_async_copy(dst_hbm.at[wid, 0], dstb[gb], si[gb]).wait()
      if j == 2:
        @pl.when(g + 1 < NG)
        def _():
          pltpu.async_copy(src_hbm.at[wid, g + 1], srcb[1 - gb], si[1 - gb])
          pltpu.async_copy(dst_hbm.at[wid, g + 1], dstb[1 - gb], si[1 - gb])
      pltpu.async_copy(h_hbm.at[srcb[gb].at[j]], rows[b], sg0).wait()
      pltpu.async_copy(rows[b], acc_sh.at[dstb[gb].at[j]], ss[b], add=True)

    @pl.loop(0, NG, step=2)
    def _(go):
      for gb in (0, 1):
        for j in range(G):
          chunk(go + gb, gb, j)

    pltpu.make_async_copy(h_hbm.at[srcb0.at[0]], rows0, ss0).wait()
    pltpu.make_async_copy(h_hbm.at[srcb0.at[0]], rows1, ss1).wait()

    plsc.subcore_barrier()
    out_r = cid * NP + row0

    @pl.loop(0, RPW, step=K)
    def _(j):
      pltpu.sync_copy(acc_sh.at[pl.ds(row0 + j, K)], rows0)
      pltpu.sync_copy(rows0, agg_out.at[pl.ds(out_r + j, K)])

  return body(h, src4, dst4, zeros_rows).reshape(NC, NP, D)


def _sc_deg(dst3, zeros_rows, ones_rows):
  NQ = 8

  @functools.partial(
      pl.kernel,
      mesh=_sc_mesh(),
      out_type=jax.ShapeDtypeStruct((NC * NP, D), jnp.float32),
      scratch_types=[
          pltpu.VMEM((NCH, K), jnp.int32),
          pltpu.VMEM((K, D), jnp.float32),
          pltpu.VMEM_SHARED((NP, D), jnp.float32),
          pltpu.SemaphoreType.DMA,
          pltpu.SemaphoreType.DMA,
      ],
  )
  def body(dst_hbm, zr_hbm, ones_hbm, deg_out, dst_v, rows_v, acc_sh,
           ssc, sst):
    cid = lax.axis_index("c")
    sid = lax.axis_index("s")
    wid = cid * NS + sid
    row0 = sid * RPW
    pltpu.async_copy(dst_hbm.at[wid], dst_v, sst).wait()
    pltpu.async_copy(zr_hbm, rows_v, sst).wait()

    @pl.loop(0, RPW, step=K)
    def _(j):
      pltpu.sync_copy(rows_v, acc_sh.at[pl.ds(row0 + j, K)])

    plsc.subcore_barrier()
    pltpu.async_copy(ones_hbm, rows_v, sst).wait()

    @pl.loop(0, NQ)
    def _(cc):
      pltpu.async_copy(rows_v, acc_sh.at[dst_v.at[cc]], ssc, add=True)

    @pl.loop(NQ, NCH)
    def _(cc):
      pltpu.make_async_copy(ones_hbm, rows_v, ssc).wait()
      pltpu.async_copy(rows_v, acc_sh.at[dst_v.at[cc]], ssc, add=True)

    @pl.loop(0, NQ)
    def _(cc):
      pltpu.make_async_copy(ones_hbm, rows_v, ssc).wait()

    plsc.subcore_barrier()
    out_r = cid * NP + row0

    @pl.loop(0, RPW, step=K)
    def _(j):
      pltpu.sync_copy(acc_sh.at[pl.ds(row0 + j, K)], rows_v)
      pltpu.sync_copy(rows_v, deg_out.at[pl.ds(out_r + j, K)])

  return body(dst3, zeros_rows, ones_rows).reshape(NC, NP, D)


ROWS_BLK = 1000


def _tc_layer1_body(h_ref, agg_ref, deg_ref, ws_ref, wn_ref, b_ref, out_ref):
  agg = agg_ref[0] + agg_ref[1]
  deg = deg_ref[0] + deg_ref[1]
  dinv = 1.0 / jnp.maximum(deg, 1.0)
  hs = jnp.dot(h_ref[...], ws_ref[...], preferred_element_type=jnp.float32)
  hn = jnp.dot(agg, wn_ref[...], preferred_element_type=jnp.float32)
  out_ref[...] = jnp.maximum(hs + hn * dinv + b_ref[...], 0.0)


def _tc_layer1(h, aggp, degs, W_self, W_neigh, b):
  grid = (N // ROWS_BLK,)
  return pl.pallas_call(
      _tc_layer1_body,
      grid=grid,
      in_specs=[
          pl.BlockSpec((ROWS_BLK, D), lambda i: (i, 0)),
          pl.BlockSpec((NC, ROWS_BLK, D), lambda i: (0, i, 0)),
          pl.BlockSpec((NC, ROWS_BLK, 1), lambda i: (0, i, 0)),
          pl.BlockSpec((D, D), lambda i: (0, 0)),
          pl.BlockSpec((D, D), lambda i: (0, 0)),
          pl.BlockSpec((1, D), lambda i: (0, 0)),
      ],
      out_specs=pl.BlockSpec((ROWS_BLK, D), lambda i: (i, 0)),
      out_shape=jax.ShapeDtypeStruct((N, D), jnp.float32),
  )(h, aggp, degs, W_self, W_neigh, b.reshape(1, D))


def _tc_layer2_body(h_ref, agg_ref, deg_ref, ws_ref, wn_ref, b_ref,
                    wr0_ref, br0_ref, wr1_ref, br1_ref, out_ref):
  agg = agg_ref[0] + agg_ref[1]
  deg = deg_ref[0] + deg_ref[1]
  dinv = 1.0 / jnp.maximum(deg, 1.0)
  hs = jnp.dot(h_ref[...], ws_ref[...], preferred_element_type=jnp.float32)
  hn = jnp.dot(agg, wn_ref[...], preferred_element_type=jnp.float32)
  h2 = jnp.maximum(hs + hn * dinv + b_ref[...], 0.0)
  r = jnp.maximum(
      jnp.dot(h2, wr0_ref[...], preferred_element_type=jnp.float32)
      + br0_ref[...], 0.0)
  out_ref[...] = (
      jnp.dot(r, wr1_ref[...], preferred_element_type=jnp.float32)
      + br1_ref[...])


def _tc_layer2(h, aggp, degs, W_self, W_neigh, b, Wr0, br0, Wr1, br1):
  grid = (N // ROWS_BLK,)
  H1 = Wr0.shape[1]
  return pl.pallas_call(
      _tc_layer2_body,
      grid=grid,
      in_specs=[
          pl.BlockSpec((ROWS_BLK, D), lambda i: (i, 0)),
          pl.BlockSpec((NC, ROWS_BLK, D), lambda i: (0, i, 0)),
          pl.BlockSpec((NC, ROWS_BLK, 1), lambda i: (0, i, 0)),
          pl.BlockSpec((D, D), lambda i: (0, 0)),
          pl.BlockSpec((D, D), lambda i: (0, 0)),
          pl.BlockSpec((1, D), lambda i: (0, 0)),
          pl.BlockSpec((D, H1), lambda i: (0, 0)),
          pl.BlockSpec((1, H1), lambda i: (0, 0)),
          pl.BlockSpec((H1, 1), lambda i: (0, 0)),
          pl.BlockSpec((1, 1), lambda i: (0, 0)),
      ],
      out_specs=pl.BlockSpec((ROWS_BLK, 1), lambda i: (i, 0)),
      out_shape=jax.ShapeDtypeStruct((N, 1), jnp.float32),
  )(h, aggp, degs, W_self, W_neigh, b.reshape(1, D),
    Wr0, br0.reshape(1, H1), Wr1, br1.reshape(1, 1))


def kernel(x, edge_index, W_self0, W_neigh0, b0, W_self1, W_neigh1, b1,
           Wr0, br0, Wr1, br1):
  ei = edge_index.astype(jnp.int32)
  src4 = jnp.concatenate(
      [ei[0], jnp.zeros((EPAD - E,), jnp.int32)]).reshape(NW, NG, G, K)
  pad_dst = N + (jnp.arange(EPAD - E, dtype=jnp.int32) % (NP - N))
  dst4 = jnp.concatenate([ei[1], pad_dst]).reshape(NW, NG, G, K)
  dst3 = dst4.reshape(NW, NCH, K)
  zeros_rows = jnp.zeros((K, D), jnp.float32)
  ones_rows = jnp.ones((K, D), jnp.float32)

  degp = _sc_deg(dst3, zeros_rows, ones_rows)
  degs = degp[:, :, :1]
  x_dep = x + 0.0 * degs[0, :N]
  agg0 = _sc_agg(x_dep, src4, dst4, zeros_rows)
  h1 = _tc_layer1(x, agg0, degs, W_self0, W_neigh0, b0)
  agg1 = _sc_agg(h1, src4, dst4, zeros_rows)
  return _tc_layer2(h1, agg1, degs, W_self1, W_neigh1, b1, Wr0, br0, Wr1, br1)


# speedup vs baseline: 1.8839x; 1.8812x over previous
import functools

import jax
import jax.numpy as jnp
from jax import lax
from jax.experimental import pallas as pl
from jax.experimental.pallas import tpu as pltpu
from jax.experimental.pallas import tpu_sc as plsc

N = 10000
E = 320000
D = 128
NC = 2
NS = 16
NW = NC * NS
K = 128
NCH = 80
EPWP = NCH * K
EPAD = NW * EPWP
NP = 10240
RPW = NP // NS


@functools.cache
def _sc_mesh():
  return plsc.VectorSubcoreMesh(core_axis_name="c", subcore_axis_name="s")


KA = 80
NCHA = 125


def _sc_agg(h, src, dst, zeros_rows):
  @functools.partial(
      pl.kernel,
      mesh=_sc_mesh(),
      out_type=jax.ShapeDtypeStruct((NC * NP, D), jnp.float32),
      scratch_types=[
          pltpu.VMEM((KA,), jnp.int32),
          pltpu.VMEM((KA,), jnp.int32),
          pltpu.VMEM((KA,), jnp.int32),
          pltpu.VMEM((KA,), jnp.int32),
          pltpu.VMEM((KA, D), jnp.float32),
          pltpu.VMEM((KA, D), jnp.float32),
          pltpu.VMEM((K, D), jnp.float32),
          pltpu.VMEM_SHARED((NP, D), jnp.float32),
          pltpu.SemaphoreType.DMA,
          pltpu.SemaphoreType.DMA,
          pltpu.SemaphoreType.DMA,
          pltpu.SemaphoreType.DMA,
      ],
  )
  def body(h_hbm, src_hbm, dst_hbm, zr_hbm, agg_out, srcv0, srcv1,
           dstv0, dstv1, rows0, rows1, zstage, acc_sh, sgt, ss0, ss1, sst):
    cid = lax.axis_index("c")
    sid = lax.axis_index("s")
    row0 = sid * RPW
    srcv = (srcv0, srcv1)
    dstv = (dstv0, dstv1)
    rows = (rows0, rows1)
    ss = (ss0, ss1)

    pltpu.async_copy(zr_hbm, zstage, sst).wait()

    @pl.loop(0, RPW, step=K)
    def _(j):
      pltpu.sync_copy(zstage, acc_sh.at[pl.ds(row0 + j, K)])

    plsc.subcore_barrier()
    base = cid * (E // NC) + sid * (E // NW)

    def chunk(cc, b, first):
      if not first:
        pltpu.make_async_copy(zr_hbm.at[pl.ds(0, KA)], rows[b], ss[b]).wait()
      pltpu.sync_copy(src_hbm.at[pl.ds(base + cc * KA, KA)], srcv[b])
      pltpu.async_copy(h_hbm.at[srcv[b]], rows[b], sgt).wait()
      pltpu.sync_copy(dst_hbm.at[pl.ds(base + cc * KA, KA)], dstv[b])
      pltpu.async_copy(rows[b], acc_sh.at[dstv[b]], ss[b], add=True)

    chunk(0, 0, True)
    chunk(1, 1, True)

    @pl.loop(2, NCHA - 1, step=2)
    def _(c):
      for bb in (0, 1):
        chunk(c + bb, bb, False)

    chunk(NCHA - 1, 0, False)

    pltpu.make_async_copy(zr_hbm.at[pl.ds(0, KA)], rows1, ss1).wait()
    pltpu.make_async_copy(zr_hbm.at[pl.ds(0, KA)], rows0, ss0).wait()

    plsc.subcore_barrier()
    out_r = cid * NP + row0

    @pl.loop(0, RPW, step=K)
    def _(j):
      pltpu.sync_copy(acc_sh.at[pl.ds(row0 + j, K)], zstage)
      pltpu.sync_copy(zstage, agg_out.at[pl.ds(out_r + j, K)])

  return body(h, src, dst, zeros_rows).reshape(NC, NP, D)


def _sc_deg(dst3, zeros_rows, ones_rows):
  NQ = 8

  @functools.partial(
      pl.kernel,
      mesh=_sc_mesh(),
      out_type=jax.ShapeDtypeStruct((NC * NP, D), jnp.float32),
      scratch_types=[
          pltpu.VMEM((NCH, K), jnp.int32),
          pltpu.VMEM((K, D), jnp.float32),
          pltpu.VMEM_SHARED((NP, D), jnp.float32),
          pltpu.SemaphoreType.DMA,
          pltpu.SemaphoreType.DMA,
      ],
  )
  def body(dst_hbm, zr_hbm, ones_hbm, deg_out, dst_v, rows_v, acc_sh,
           ssc, sst):
    cid = lax.axis_index("c")
    sid = lax.axis_index("s")
    wid = cid * NS + sid
    row0 = sid * RPW
    pltpu.async_copy(dst_hbm.at[wid], dst_v, sst).wait()
    pltpu.async_copy(zr_hbm, rows_v, sst).wait()

    @pl.loop(0, RPW, step=K)
    def _(j):
      pltpu.sync_copy(rows_v, acc_sh.at[pl.ds(row0 + j, K)])

    plsc.subcore_barrier()
    pltpu.async_copy(ones_hbm, rows_v, sst).wait()

    @pl.loop(0, NQ)
    def _(cc):
      pltpu.async_copy(rows_v, acc_sh.at[dst_v.at[cc]], ssc, add=True)

    @pl.loop(NQ, NCH)
    def _(cc):
      pltpu.make_async_copy(ones_hbm, rows_v, ssc).wait()
      pltpu.async_copy(rows_v, acc_sh.at[dst_v.at[cc]], ssc, add=True)

    @pl.loop(0, NQ)
    def _(cc):
      pltpu.make_async_copy(ones_hbm, rows_v, ssc).wait()

    plsc.subcore_barrier()
    out_r = cid * NP + row0

    @pl.loop(0, RPW, step=K)
    def _(j):
      pltpu.sync_copy(acc_sh.at[pl.ds(row0 + j, K)], rows_v)
      pltpu.sync_copy(rows_v, deg_out.at[pl.ds(out_r + j, K)])

  return body(dst3, zeros_rows, ones_rows).reshape(NC, NP, D)


ROWS_BLK = 1000


def _tc_layer1_body(h_ref, agg_ref, deg_ref, ws_ref, wn_ref, b_ref, out_ref):
  agg = agg_ref[0] + agg_ref[1]
  deg = deg_ref[0] + deg_ref[1]
  dinv = 1.0 / jnp.maximum(deg, 1.0)
  hs = jnp.dot(h_ref[...], ws_ref[...], preferred_element_type=jnp.float32)
  hn = jnp.dot(agg, wn_ref[...], preferred_element_type=jnp.float32)
  out_ref[...] = jnp.maximum(hs + hn * dinv + b_ref[...], 0.0)


def _tc_layer1(h, aggp, degs, W_self, W_neigh, b):
  grid = (N // ROWS_BLK,)
  return pl.pallas_call(
      _tc_layer1_body,
      grid=grid,
      in_specs=[
          pl.BlockSpec((ROWS_BLK, D), lambda i: (i, 0)),
          pl.BlockSpec((NC, ROWS_BLK, D), lambda i: (0, i, 0)),
          pl.BlockSpec((NC, ROWS_BLK, 1), lambda i: (0, i, 0)),
          pl.BlockSpec((D, D), lambda i: (0, 0)),
          pl.BlockSpec((D, D), lambda i: (0, 0)),
          pl.BlockSpec((1, D), lambda i: (0, 0)),
      ],
      out_specs=pl.BlockSpec((ROWS_BLK, D), lambda i: (i, 0)),
      out_shape=jax.ShapeDtypeStruct((N, D), jnp.float32),
  )(h, aggp, degs, W_self, W_neigh, b.reshape(1, D))


def _tc_layer2_body(h_ref, agg_ref, deg_ref, ws_ref, wn_ref, b_ref,
                    wr0_ref, br0_ref, wr1_ref, br1_ref, out_ref):
  agg = agg_ref[0] + agg_ref[1]
  deg = deg_ref[0] + deg_ref[1]
  dinv = 1.0 / jnp.maximum(deg, 1.0)
  hs = jnp.dot(h_ref[...], ws_ref[...], preferred_element_type=jnp.float32)
  hn = jnp.dot(agg, wn_ref[...], preferred_element_type=jnp.float32)
  h2 = jnp.maximum(hs + hn * dinv + b_ref[...], 0.0)
  r = jnp.maximum(
      jnp.dot(h2, wr0_ref[...], preferred_element_type=jnp.float32)
      + br0_ref[...], 0.0)
  out_ref[...] = (
      jnp.dot(r, wr1_ref[...], preferred_element_type=jnp.float32)
      + br1_ref[...])


def _tc_layer2(h, aggp, degs, W_self, W_neigh, b, Wr0, br0, Wr1, br1):
  grid = (N // ROWS_BLK,)
  H1 = Wr0.shape[1]
  return pl.pallas_call(
      _tc_layer2_body,
      grid=grid,
      in_specs=[
          pl.BlockSpec((ROWS_BLK, D), lambda i: (i, 0)),
          pl.BlockSpec((NC, ROWS_BLK, D), lambda i: (0, i, 0)),
          pl.BlockSpec((NC, ROWS_BLK, 1), lambda i: (0, i, 0)),
          pl.BlockSpec((D, D), lambda i: (0, 0)),
          pl.BlockSpec((D, D), lambda i: (0, 0)),
          pl.BlockSpec((1, D), lambda i: (0, 0)),
          pl.BlockSpec((D, H1), lambda i: (0, 0)),
          pl.BlockSpec((1, H1), lambda i: (0, 0)),
          pl.BlockSpec((H1, 1), lambda i: (0, 0)),
          pl.BlockSpec((1, 1), lambda i: (0, 0)),
      ],
      out_specs=pl.BlockSpec((ROWS_BLK, 1), lambda i: (i, 0)),
      out_shape=jax.ShapeDtypeStruct((N, 1), jnp.float32),
  )(h, aggp, degs, W_self, W_neigh, b.reshape(1, D),
    Wr0, br0.reshape(1, H1), Wr1, br1.reshape(1, 1))


def kernel(x, edge_index, W_self0, W_neigh0, b0, W_self1, W_neigh1, b1,
           Wr0, br0, Wr1, br1):
  ei = edge_index.astype(jnp.int32)
  src = ei[0]
  dst = ei[1]
  pad_dst = N + (jnp.arange(EPAD - E, dtype=jnp.int32) % (NP - N))
  dst3 = jnp.concatenate([dst, pad_dst]).reshape(NW, NCH, K)
  zeros_rows = jnp.zeros((K, D), jnp.float32)
  ones_rows = jnp.ones((K, D), jnp.float32)

  degp = _sc_deg(dst3, zeros_rows, ones_rows)
  degs = degp[:, :, :1]
  x_dep = x + 0.0 * degs[0, :N]
  agg0 = _sc_agg(x_dep, src, dst, zeros_rows)
  h1 = _tc_layer1(x, agg0, degs, W_self0, W_neigh0, b0)
  agg1 = _sc_agg(h1, src, dst, zeros_rows)
  return _tc_layer2(h1, agg1, degs, W_self1, W_neigh1, b1, Wr0, br0, Wr1, br1)


# speedup vs baseline: 2.7115x; 1.4393x over previous
import functools

import jax
import jax.numpy as jnp
from jax import lax
from jax.experimental import pallas as pl
from jax.experimental.pallas import tpu as pltpu
from jax.experimental.pallas import tpu_sc as plsc

N = 10000
E = 320000
D = 128
NC = 2
NS = 16
NW = NC * NS
K = 128
NCH = 80
EPWP = NCH * K
EPAD = NW * EPWP
NP = 10240
RPW = NP // NS


@functools.cache
def _sc_mesh():
  return plsc.VectorSubcoreMesh(core_axis_name="c", subcore_axis_name="s")


KA = 80
NCHA = 125


def _sc_agg(h, src, dst, zeros_rows):
  @functools.partial(
      pl.kernel,
      mesh=_sc_mesh(),
      out_type=jax.ShapeDtypeStruct((NC * NP, D), jnp.float32),
      scratch_types=(
          [pltpu.VMEM((KA,), jnp.int32) for _ in range(4)]
          + [pltpu.VMEM((KA,), jnp.int32) for _ in range(4)]
          + [
              pltpu.VMEM((KA, D), jnp.float32),
              pltpu.VMEM((KA, D), jnp.float32),
              pltpu.VMEM((K, D), jnp.float32),
              pltpu.VMEM_SHARED((NP, D), jnp.float32),
              pltpu.SemaphoreType.DMA,
              pltpu.SemaphoreType.DMA,
              pltpu.SemaphoreType.DMA,
          ]
          + [pltpu.SemaphoreType.DMA for _ in range(4)]
          + [pltpu.SemaphoreType.DMA]
      ),
  )
  def body(h_hbm, src_hbm, dst_hbm, zr_hbm, agg_out,
           sv0, sv1, sv2, sv3, dv0, dv1, dv2, dv3,
           rows0, rows1, zstage, acc_sh, sgt, ss0, ss1,
           si0, si1, si2, si3, sst):
    cid = lax.axis_index("c")
    sid = lax.axis_index("s")
    row0 = sid * RPW
    srcv = (sv0, sv1, sv2, sv3)
    dstv = (dv0, dv1, dv2, dv3)
    rows = (rows0, rows1)
    ss = (ss0, ss1)
    si = (si0, si1, si2, si3)

    pltpu.async_copy(zr_hbm, zstage, sst).wait()

    @pl.loop(0, RPW, step=K)
    def _(j):
      pltpu.sync_copy(zstage, acc_sh.at[pl.ds(row0 + j, K)])

    plsc.subcore_barrier()
    base = cid * (E // NC) + sid * (E // NW)

    def idx_load(cc, q):
      pltpu.async_copy(src_hbm.at[pl.ds(base + cc * KA, KA)], srcv[q], si[q])
      pltpu.async_copy(dst_hbm.at[pl.ds(base + cc * KA, KA)], dstv[q], si[q])

    def chunk(cc, q, b, first):
      if not first:
        pltpu.make_async_copy(zr_hbm.at[pl.ds(0, KA)], rows[b], ss[b]).wait()
      q2 = (q + 2) % 4
      if isinstance(cc, int):
        if cc + 2 < NCHA:
          idx_load(cc + 2, q2)
      else:
        @pl.when(cc + 2 < NCHA)
        def _():
          idx_load(cc + 2, q2)
      pltpu.make_async_copy(src_hbm.at[pl.ds(base, KA)], srcv[q],
                            si[q]).wait()
      pltpu.make_async_copy(src_hbm.at[pl.ds(base, KA)], dstv[q],
                            si[q]).wait()
      pltpu.async_copy(h_hbm.at[srcv[q]], rows[b], sgt).wait()
      pltpu.async_copy(rows[b], acc_sh.at[dstv[q]], ss[b], add=True)

    idx_load(0, 0)
    idx_load(1, 1)
    chunk(0, 0, 0, True)
    chunk(1, 1, 1, True)
    chunk(2, 2, 0, False)
    chunk(3, 3, 1, False)

    @pl.loop(4, NCHA - 1, step=4)
    def _(c):
      for bb in (0, 1, 2, 3):
        chunk(c + bb, bb, bb % 2, False)

    chunk(NCHA - 1, 0, 0, False)

    pltpu.make_async_copy(zr_hbm.at[pl.ds(0, KA)], rows1, ss1).wait()
    pltpu.make_async_copy(zr_hbm.at[pl.ds(0, KA)], rows0, ss0).wait()

    plsc.subcore_barrier()
    out_r = cid * NP + row0

    @pl.loop(0, RPW, step=K)
    def _(j):
      pltpu.sync_copy(acc_sh.at[pl.ds(row0 + j, K)], zstage)
      pltpu.sync_copy(zstage, agg_out.at[pl.ds(out_r + j, K)])

  return body(h, src, dst, zeros_rows).reshape(NC, NP, D)


def _sc_deg(dst3, zeros_rows, ones_rows):
  NQ = 8

  @functools.partial(
      pl.kernel,
      mesh=_sc_mesh(),
      out_type=jax.ShapeDtypeStruct((NC * NP, D), jnp.float32),
      scratch_types=[
          pltpu.VMEM((NCH, K), jnp.int32),
          pltpu.VMEM((K, D), jnp.float32),
          pltpu.VMEM_SHARED((NP, D), jnp.float32),
          pltpu.SemaphoreType.DMA,
          pltpu.SemaphoreType.DMA,
      ],
  )
  def body(dst_hbm, zr_hbm, ones_hbm, deg_out, dst_v, rows_v, acc_sh,
           ssc, sst):
    cid = lax.axis_index("c")
    sid = lax.axis_index("s")
    wid = cid * NS + sid
    row0 = sid * RPW
    pltpu.async_copy(dst_hbm.at[wid], dst_v, sst).wait()
    pltpu.async_copy(zr_hbm, rows_v, sst).wait()

    @pl.loop(0, RPW, step=K)
    def _(j):
      pltpu.sync_copy(rows_v, acc_sh.at[pl.ds(row0 + j, K)])

    plsc.subcore_barrier()
    pltpu.async_copy(ones_hbm, rows_v, sst).wait()

    @pl.loop(0, NQ)
    def _(cc):
      pltpu.async_copy(rows_v, acc_sh.at[dst_v.at[cc]], ssc, add=True)

    @pl.loop(NQ, NCH)
    def _(cc):
      pltpu.make_async_copy(ones_hbm, rows_v, ssc).wait()
      pltpu.async_copy(rows_v, acc_sh.at[dst_v.at[cc]], ssc, add=True)

    @pl.loop(0, NQ)
    def _(cc):
      pltpu.make_async_copy(ones_hbm, rows_v, ssc).wait()

    plsc.subcore_barrier()
    out_r = cid * NP + row0

    @pl.loop(0, RPW, step=K)
    def _(j):
      pltpu.sync_copy(acc_sh.at[pl.ds(row0 + j, K)], rows_v)
      pltpu.sync_copy(rows_v, deg_out.at[pl.ds(out_r + j, K)])

  return body(dst3, zeros_rows, ones_rows).reshape(NC, NP, D)


ROWS_BLK = 1000


def _tc_layer1_body(h_ref, agg_ref, deg_ref, ws_ref, wn_ref, b_ref, out_ref):
  agg = agg_ref[0] + agg_ref[1]
  deg = deg_ref[0] + deg_ref[1]
  dinv = 1.0 / jnp.maximum(deg, 1.0)
  hs = jnp.dot(h_ref[...], ws_ref[...], preferred_element_type=jnp.float32)
  hn = jnp.dot(agg, wn_ref[...], preferred_element_type=jnp.float32)
  out_ref[...] = jnp.maximum(hs + hn * dinv + b_ref[...], 0.0)


def _tc_layer1(h, aggp, degs, W_self, W_neigh, b):
  grid = (N // ROWS_BLK,)
  return pl.pallas_call(
      _tc_layer1_body,
      grid=grid,
      in_specs=[
          pl.BlockSpec((ROWS_BLK, D), lambda i: (i, 0)),
          pl.BlockSpec((NC, ROWS_BLK, D), lambda i: (0, i, 0)),
          pl.BlockSpec((NC, ROWS_BLK, 1), lambda i: (0, i, 0)),
          pl.BlockSpec((D, D), lambda i: (0, 0)),
          pl.BlockSpec((D, D), lambda i: (0, 0)),
          pl.BlockSpec((1, D), lambda i: (0, 0)),
      ],
      out_specs=pl.BlockSpec((ROWS_BLK, D), lambda i: (i, 0)),
      out_shape=jax.ShapeDtypeStruct((N, D), jnp.float32),
  )(h, aggp, degs, W_self, W_neigh, b.reshape(1, D))


def _tc_layer2_body(h_ref, agg_ref, deg_ref, ws_ref, wn_ref, b_ref,
                    wr0_ref, br0_ref, wr1_ref, br1_ref, out_ref):
  agg = agg_ref[0] + agg_ref[1]
  deg = deg_ref[0] + deg_ref[1]
  dinv = 1.0 / jnp.maximum(deg, 1.0)
  hs = jnp.dot(h_ref[...], ws_ref[...], preferred_element_type=jnp.float32)
  hn = jnp.dot(agg, wn_ref[...], preferred_element_type=jnp.float32)
  h2 = jnp.maximum(hs + hn * dinv + b_ref[...], 0.0)
  r = jnp.maximum(
      jnp.dot(h2, wr0_ref[...], preferred_element_type=jnp.float32)
      + br0_ref[...], 0.0)
  out_ref[...] = (
      jnp.dot(r, wr1_ref[...], preferred_element_type=jnp.float32)
      + br1_ref[...])


def _tc_layer2(h, aggp, degs, W_self, W_neigh, b, Wr0, br0, Wr1, br1):
  grid = (N // ROWS_BLK,)
  H1 = Wr0.shape[1]
  return pl.pallas_call(
      _tc_layer2_body,
      grid=grid,
      in_specs=[
          pl.BlockSpec((ROWS_BLK, D), lambda i: (i, 0)),
          pl.BlockSpec((NC, ROWS_BLK, D), lambda i: (0, i, 0)),
          pl.BlockSpec((NC, ROWS_BLK, 1), lambda i: (0, i, 0)),
          pl.BlockSpec((D, D), lambda i: (0, 0)),
          pl.BlockSpec((D, D), lambda i: (0, 0)),
          pl.BlockSpec((1, D), lambda i: (0, 0)),
          pl.BlockSpec((D, H1), lambda i: (0, 0)),
          pl.BlockSpec((1, H1), lambda i: (0, 0)),
          pl.BlockSpec((H1, 1), lambda i: (0, 0)),
          pl.BlockSpec((1, 1), lambda i: (0, 0)),
      ],
      out_specs=pl.BlockSpec((ROWS_BLK, 1), lambda i: (i, 0)),
      out_shape=jax.ShapeDtypeStruct((N, 1), jnp.float32),
  )(h, aggp, degs, W_self, W_neigh, b.reshape(1, D),
    Wr0, br0.reshape(1, H1), Wr1, br1.reshape(1, 1))


def kernel(x, edge_index, W_self0, W_neigh0, b0, W_self1, W_neigh1, b1,
           Wr0, br0, Wr1, br1):
  ei = edge_index.astype(jnp.int32)
  src = ei[0]
  dst = ei[1]
  pad_dst = N + (jnp.arange(EPAD - E, dtype=jnp.int32) % (NP - N))
  dst3 = jnp.concatenate([dst, pad_dst]).reshape(NW, NCH, K)
  zeros_rows = jnp.zeros((K, D), jnp.float32)
  ones_rows = jnp.ones((K, D), jnp.float32)

  degp = _sc_deg(dst3, zeros_rows, ones_rows)
  degs = degp[:, :, :1]
  x_dep = x + 0.0 * degs[0, :N]
  agg0 = _sc_agg(x_dep, src, dst, zeros_rows)
  h1 = _tc_layer1(x, agg0, degs, W_self0, W_neigh0, b0)
  agg1 = _sc_agg(h1, src, dst, zeros_rows)
  return _tc_layer2(h1, agg1, degs, W_self1, W_neigh1, b1, Wr0, br0, Wr1, br1)
